# baseline (speedup 1.0000x reference)
import functools

import jax
import jax.numpy as jnp
from jax import lax
from jax.experimental import pallas as pl
from jax.experimental.pallas import tpu as pltpu
from jax.experimental.pallas import tpu_sc as plsc

N = 50000
M = 16
AFL = 64
EPS = 1e-5

_NC = 2
_NS = 16
_NW = _NC * _NS

_BA = 400
_NBLK = N // _BA
_BAC = 5000


def _softplus(x):
    return jnp.maximum(x, 0.0) + jnp.log1p(jnp.exp(-jnp.abs(x)))


def _rsqrt(x):
    r = lax.rsqrt(x)
    return r * (1.5 - 0.5 * x * r * r)


def _recip(d):
    r = 1.0 / d
    return r * (2.0 - d * r)


def _bdot(a, b):
    return jnp.dot(a.astype(jnp.bfloat16), b.astype(jnp.bfloat16),
                   preferred_element_type=jnp.float32)


def _emb_body(a_ref, w_ref, b_ref, wn_ref, x_ref, yn_ref):
    x = _bdot(a_ref[...], w_ref[...]) + b_ref[...]
    x_ref[...] = x
    yn_ref[...] = _bdot(x, wn_ref[...])


def _embed(atom_fea, w, b, wnbr):
    return pl.pallas_call(
        _emb_body,
        grid=(N // _BAC,),
        in_specs=[
            pl.BlockSpec((_BAC, 128), lambda i: (i, 0)),
            pl.BlockSpec((128, AFL), lambda i: (0, 0)),
            pl.BlockSpec((1, AFL), lambda i: (0, 0)),
            pl.BlockSpec((AFL, 2 * AFL), lambda i: (0, 0)),
        ],
        out_specs=[
            pl.BlockSpec((_BAC, AFL), lambda i: (i, 0)),
            pl.BlockSpec((_BAC, 2 * AFL), lambda i: (i, 0)),
        ],
        out_shape=[
            jax.ShapeDtypeStruct((N, AFL), jnp.float32),
            jax.ShapeDtypeStruct((N, 2 * AFL), jnp.float32),
        ],
    )(atom_fea, w, b.reshape(1, AFL), wnbr)


_B_TOT = N * M
_BPW = _B_TOT // _NW
_CH = 200
_NCH = _BPW // _CH


@functools.cache
def _sc_gather_kernel():
    @functools.partial(
        pl.kernel,
        mesh=plsc.VectorSubcoreMesh(core_axis_name="c", subcore_axis_name="s"),
        out_type=jax.ShapeDtypeStruct((_B_TOT, 2 * AFL), jnp.float32),
        scratch_types=[
            pltpu.VMEM((_BPW,), jnp.int32),
            pltpu.VMEM((_CH, 2 * AFL), jnp.float32),
            pltpu.VMEM((_CH, 2 * AFL), jnp.float32),
            pltpu.SemaphoreType.DMA,
            pltpu.SemaphoreType.DMA,
        ],
    )
    def _sc_gather(table_hbm, idx_hbm, out_hbm, idx_v, rows_a, rows_b, sem_a, sem_b):
        wid = lax.axis_index("s") * _NC + lax.axis_index("c")
        base = wid * _BPW
        pltpu.sync_copy(idx_hbm.at[pl.ds(base, _BPW)], idx_v)

        def mk(t, rows, sem):
            return pltpu.make_async_copy(
                table_hbm.at[idx_v.at[pl.ds(t * _CH, _CH)]], rows, sem)

        mk(0, rows_a, sem_a).start()

        def chunk(t, carry):
            even = t % 2 == 0

            @pl.when(even)
            def _():
                mk(t, rows_a, sem_a).wait()

                @pl.when(t + 1 < _NCH)
                def _():
                    mk(t + 1, rows_b, sem_b).start()

                pltpu.sync_copy(rows_a, out_hbm.at[pl.ds(base + t * _CH, _CH)])

            @pl.when(jnp.logical_not(even))
            def _():
                mk(t, rows_b, sem_b).wait()

                @pl.when(t + 1 < _NCH)
                def _():
                    mk(t + 1, rows_a, sem_a).start()

                pltpu.sync_copy(rows_b, out_hbm.at[pl.ds(base + t * _CH, _CH)])

            return carry

        lax.fori_loop(0, _NCH, chunk, 0)

    return _sc_gather


def _stats_body(yng_ref, nbr_ref, x_ref, wedge_ref, wself_ref, b_ref,
                st_ref, gs8, gq8, cs8, ys8, yq8):
    i = pl.program_id(0)

    @pl.when(i == 0)
    def _():
        for r in (gs8, gq8, cs8, ys8, yq8):
            r[...] = jnp.zeros_like(r)

    g = yng_ref[...] + _bdot(nbr_ref[...], wedge_ref[...])
    ys = _bdot(x_ref[...], wself_ref[...]) + b_ref[...]
    gs8[...] += jnp.sum(g.reshape(_BA * 2, 8, 2 * AFL), axis=0)
    gq8[...] += jnp.sum((g * g).reshape(_BA * 2, 8, 2 * AFL), axis=0)
    sg = jnp.sum(g.reshape(_BA, M, 2 * AFL), axis=1)
    cs8[...] += jnp.sum((ys * sg).reshape(_BA // 8, 8, 2 * AFL), axis=0)
    ys8[...] += jnp.sum(ys.reshape(_BA // 8, 8, 2 * AFL), axis=0)
    yq8[...] += jnp.sum((ys * ys).reshape(_BA // 8, 8, 2 * AFL), axis=0)

    @pl.when(i == _NBLK - 1)
    def _():
        ssum = jnp.sum(gs8[...], axis=0, keepdims=True) + M * jnp.sum(
            ys8[...], axis=0, keepdims=True)
        ssq = (jnp.sum(gq8[...], axis=0, keepdims=True)
               + 2.0 * jnp.sum(cs8[...], axis=0, keepdims=True)
               + M * jnp.sum(yq8[...], axis=0, keepdims=True))
        st_ref[0:1, :] = ssum
        st_ref[1:2, :] = ssq


def _conv_stats(yng, nbr2, x, wedge, wself, b):
    return pl.pallas_call(
        _stats_body,
        grid=(_NBLK,),
        in_specs=[
            pl.BlockSpec((_BA * M, 2 * AFL), lambda i: (i, 0)),
            pl.BlockSpec((_BA * M, M), lambda i: (i, 0)),
            pl.BlockSpec((_BA, AFL), lambda i: (i, 0)),
            pl.BlockSpec((M, 2 * AFL), lambda i: (0, 0)),
            pl.BlockSpec((AFL, 2 * AFL), lambda i: (0, 0)),
            pl.BlockSpec((1, 2 * AFL), lambda i: (0, 0)),
        ],
        out_specs=pl.BlockSpec((2, 2 * AFL), lambda i: (0, 0)),
        out_shape=jax.ShapeDtypeStruct((2, 2 * AFL), jnp.float32),
        scratch_shapes=[
            pltpu.VMEM((8, 2 * AFL), jnp.float32),
            pltpu.VMEM((8, 2 * AFL), jnp.float32),
            pltpu.VMEM((8, 2 * AFL), jnp.float32),
            pltpu.VMEM((8, 2 * AFL), jnp.float32),
            pltpu.VMEM((8, 2 * AFL), jnp.float32),
        ],
        compiler_params=pltpu.CompilerParams(
            dimension_semantics=("arbitrary",),
        ),
    )(yng, nbr2, x, wedge, wself, b.reshape(1, 2 * AFL))


_STRIP = 128
_NSTRIP = _BA * M // _STRIP


def _gate_body(yng_ref, nbr_ref, x_ref, wedge_ref, wself_ref, b_ref,
               st_ref, g1_ref, b1_ref, ns_ref, st2_ref, s28, sq8, ys_scr):
    i = pl.program_id(0)

    @pl.when(i == 0)
    def _():
        s28[...] = jnp.zeros_like(s28)
        sq8[...] = jnp.zeros_like(sq8)

    cnt = float(N * M)
    mu = st_ref[0:1, :] / cnt
    var = st_ref[1:2, :] / cnt - mu * mu
    sc = g1_ref[...] * _rsqrt(var + EPS)
    sh = b1_ref[...] - mu * sc

    ys_scr[...] = _bdot(x_ref[...], wself_ref[...]) + b_ref[...]
    wedge_f = wedge_ref[...]

    def strip(s, carry):
        r0 = s * _STRIP
        a0 = s * (_STRIP // M)
        e = _bdot(nbr_ref[pl.ds(r0, _STRIP), :], wedge_f)
        tg = yng_ref[pl.ds(r0, _STRIP), :] + e
        ysb = ys_scr[pl.ds(a0, _STRIP // M), :]
        gtd = (tg.reshape(_STRIP // M, M, 2 * AFL)
               + ysb[:, None, :]).reshape(_STRIP, 2 * AFL)
        t = gtd * sc + sh
        u = jnp.exp(-jnp.abs(t))
        sig = jnp.where(t >= 0.0, 1.0, u) * _recip(1.0 + u)
        tr = pltpu.roll(t, 64, 1)
        sp = jnp.maximum(tr, 0.0) + jnp.log(1.0 + pltpu.roll(u, 64, 1))
        gate = sig * sp
        ns128 = jnp.sum(gate.reshape(_STRIP // M, M, 2 * AFL), axis=1)
        ns_ref[pl.ds(a0, _STRIP // M), :] = ns128[:, :AFL]
        s28[...] += ns128
        sq8[...] += ns128 * ns128
        return carry

    lax.fori_loop(0, _NSTRIP, strip, 0)

    @pl.when(i == _NBLK - 1)
    def _():
        st2_ref[0:1, :] = jnp.sum(s28[...], axis=0, keepdims=True)[:, :AFL]
        st2_ref[1:2, :] = jnp.sum(sq8[...], axis=0, keepdims=True)[:, :AFL]


def _conv_gate(yng, nbr2, x, st, wedge, wself, b, g1, b1):
    return pl.pallas_call(
        _gate_body,
        grid=(_NBLK,),
        in_specs=[
            pl.BlockSpec((_BA * M, 2 * AFL), lambda i: (i, 0)),
            pl.BlockSpec((_BA * M, M), lambda i: (i, 0)),
            pl.BlockSpec((_BA, AFL), lambda i: (i, 0)),
            pl.BlockSpec((M, 2 * AFL), lambda i: (0, 0)),
            pl.BlockSpec((AFL, 2 * AFL), lambda i: (0, 0)),
            pl.BlockSpec((1, 2 * AFL), lambda i: (0, 0)),
            pl.BlockSpec((2, 2 * AFL), lambda i: (0, 0)),
            pl.BlockSpec((1, 2 * AFL), lambda i: (0, 0)),
            pl.BlockSpec((1, 2 * AFL), lambda i: (0, 0)),
        ],
        out_specs=[
            pl.BlockSpec((_BA, AFL), lambda i: (i, 0)),
            pl.BlockSpec((2, AFL), lambda i: (0, 0)),
        ],
        out_shape=[
            jax.ShapeDtypeStruct((N, AFL), jnp.float32),
            jax.ShapeDtypeStruct((2, AFL), jnp.float32),
        ],
        scratch_shapes=[
            pltpu.VMEM((8, 2 * AFL), jnp.float32),
            pltpu.VMEM((8, 2 * AFL), jnp.float32),
            pltpu.VMEM((_BA, 2 * AFL), jnp.float32),
        ],
        compiler_params=pltpu.CompilerParams(
            dimension_semantics=("arbitrary",),
        ),
    )(yng, nbr2, x, wedge, wself, b.reshape(1, 2 * AFL),
      st, g1.reshape(1, 2 * AFL), b1.reshape(1, 2 * AFL))


def _update_body(x_ref, ns_ref, st2_ref, g2_ref, b2_ref, wn_ref, x_out, yn_out):
    mu = st2_ref[0:1, :] / float(N)
    var = st2_ref[1:2, :] / float(N) - mu * mu
    sc = g2_ref[...] * _rsqrt(var + EPS)
    sh = b2_ref[...] - mu * sc
    x = _softplus(x_ref[...] + ns_ref[...] * sc + sh)
    x_out[...] = x
    yn_out[...] = _bdot(x, wn_ref[...])


def _update_last_body(x_ref, ns_ref, st2_ref, g2_ref, b2_ref, x_out):
    mu = st2_ref[0:1, :] / float(N)
    var = st2_ref[1:2, :] / float(N) - mu * mu
    sc = g2_ref[...] * _rsqrt(var + EPS)
    sh = b2_ref[...] - mu * sc
    x_out[...] = _softplus(x_ref[...] + ns_ref[...] * sc + sh)


def _update(x, ns, st2, g2, b2, wnbr_next):
    in_specs = [
        pl.BlockSpec((_BAC, AFL), lambda i: (i, 0)),
        pl.BlockSpec((_BAC, AFL), lambda i: (i, 0)),
        pl.BlockSpec((2, AFL), lambda i: (0, 0)),
        pl.BlockSpec((1, AFL), lambda i: (0, 0)),
        pl.BlockSpec((1, AFL), lambda i: (0, 0)),
    ]
    args = [x, ns, st2, g2.reshape(1, AFL), b2.reshape(1, AFL)]
    if wnbr_next is None:
        return pl.pallas_call(
            _update_last_body,
            grid=(N // _BAC,),
            in_specs=in_specs,
            out_specs=pl.BlockSpec((_BAC, AFL), lambda i: (i, 0)),
            out_shape=jax.ShapeDtypeStruct((N, AFL), jnp.float32),
        )(*args), None
    return pl.pallas_call(
        _update_body,
        grid=(N // _BAC,),
        in_specs=in_specs + [pl.BlockSpec((AFL, 2 * AFL), lambda i: (0, 0))],
        out_specs=[
            pl.BlockSpec((_BAC, AFL), lambda i: (i, 0)),
            pl.BlockSpec((_BAC, 2 * AFL), lambda i: (i, 0)),
        ],
        out_shape=[
            jax.ShapeDtypeStruct((N, AFL), jnp.float32),
            jax.ShapeDtypeStruct((N, 2 * AFL), jnp.float32),
        ],
    )(*args, wnbr_next)


def _head_body(xr_ref, gl_ref, w1a_ref, w1b_ref, b1_ref, wo_ref, bo_ref, o_ref):
    n0, kk = xr_ref.shape
    per = kk // AFL
    krow = lax.broadcasted_iota(jnp.int32, (kk, AFL), 0) % AFL
    fcol = lax.broadcasted_iota(jnp.int32, (kk, AFL), 1)
    sel = jnp.where(krow == fcol, 1.0 / per, 0.0)
    pooled = jnp.dot(xr_ref[...], sel, preferred_element_type=jnp.float32, precision=lax.Precision.HIGHEST)
    pf = _softplus(pooled)
    gf = _softplus(gl_ref[...])
    h = _bdot(jnp.concatenate([pf, gf], axis=1),
              jnp.concatenate([w1a_ref[...], w1b_ref[...]], axis=0)) + b1_ref[...]
    h = _softplus(h)
    hb = h.astype(jnp.bfloat16).astype(jnp.float32)
    wb = wo_ref[...].astype(jnp.bfloat16).astype(jnp.float32)
    o_ref[...] = jnp.sum(hb * wb, axis=1, keepdims=True) + bo_ref[...]


def _head(x, global_fea, n0, per, w1, b1, wo, bo):
    hfea = w1.shape[1]
    xr = x.reshape(n0, per * AFL)
    return pl.pallas_call(
        _head_body,
        out_shape=jax.ShapeDtypeStruct((n0, 1), jnp.float32),
    )(xr, global_fea, w1[:AFL], w1[AFL:], b1.reshape(1, hfea),
      wo.reshape(1, hfea), bo.reshape(1, 1))


def kernel(atom_fea, nbr_fea, nbr_fea_idx, crystal_atom_idx, atom_type,
           nbr_type, nbr_dist, pair_type, global_fea, params):
    n0, per = crystal_atom_idx.shape
    idx = nbr_fea_idx.reshape(-1).astype(jnp.int32)
    nbr2 = nbr_fea.reshape(N * M, M)
    convs = params["convs"]

    x, yn = _embed(atom_fea, params["emb_W"], params["emb_b"],
                   convs[0]["W"][AFL:2 * AFL])
    for li, p in enumerate(convs):
        w = p["W"]
        yng = _sc_gather_kernel()(yn, idx)
        st = _conv_stats(yng, nbr2, x, w[2 * AFL:], w[:AFL], p["b"])
        ns, st2 = _conv_gate(yng, nbr2, x, st, w[2 * AFL:], w[:AFL], p["b"],
                             p["bn1_g"], p["bn1_b"])
        wnbr_next = (convs[li + 1]["W"][AFL:2 * AFL]
                     if li + 1 < len(convs) else None)
        x, yn = _update(x, ns, st2, p["bn2_g"], p["bn2_b"], wnbr_next)

    return _head(x, global_fea, n0, per,
                 params["fc1_W"], params["fc1_b"],
                 params["out_W"], params["out_b"])


# speedup vs baseline: 1.9991x; 1.9991x over previous
import functools

import jax
import jax.numpy as jnp
from jax import lax
from jax.experimental import pallas as pl
from jax.experimental.pallas import tpu as pltpu
from jax.experimental.pallas import tpu_sc as plsc

N = 50000
M = 16
AFL = 64
EPS = 1e-5

_NC = 2
_NS = 16
_NW = _NC * _NS

_BA = 400
_NBLK = N // _BA
_BAC = 5000


def _softplus(x):
    return jnp.maximum(x, 0.0) + jnp.log1p(jnp.exp(-jnp.abs(x)))


def _rsqrt(x):
    r = lax.rsqrt(x)
    return r * (1.5 - 0.5 * x * r * r)


def _recip(d):
    r = 1.0 / d
    return r * (2.0 - d * r)


def _bdot(a, b):
    return jnp.dot(a.astype(jnp.bfloat16), b.astype(jnp.bfloat16),
                   preferred_element_type=jnp.float32)


def _emb_body(a_ref, w_ref, b_ref, wn_ref, x_ref, yn_ref):
    x = _bdot(a_ref[...], w_ref[...]) + b_ref[...]
    x_ref[...] = x
    yn_ref[...] = _bdot(x, wn_ref[...])


def _embed(atom_fea, w, b, wnbr):
    return pl.pallas_call(
        _emb_body,
        grid=(N // _BAC,),
        in_specs=[
            pl.BlockSpec((_BAC, 128), lambda i: (i, 0)),
            pl.BlockSpec((128, AFL), lambda i: (0, 0)),
            pl.BlockSpec((1, AFL), lambda i: (0, 0)),
            pl.BlockSpec((AFL, 2 * AFL), lambda i: (0, 0)),
        ],
        out_specs=[
            pl.BlockSpec((_BAC, AFL), lambda i: (i, 0)),
            pl.BlockSpec((_BAC, 2 * AFL), lambda i: (i, 0)),
        ],
        out_shape=[
            jax.ShapeDtypeStruct((N, AFL), jnp.float32),
            jax.ShapeDtypeStruct((N, 2 * AFL), jnp.float32),
        ],
    )(atom_fea, w, b.reshape(1, AFL), wnbr)


_B_TOT = N * M
_BPW = _B_TOT // _NW
_CH = 200
_NCH = _BPW // _CH


@functools.cache
def _sc_gather_kernel():
    @functools.partial(
        pl.kernel,
        mesh=plsc.VectorSubcoreMesh(core_axis_name="c", subcore_axis_name="s"),
        out_type=jax.ShapeDtypeStruct((_B_TOT, 2 * AFL), jnp.float32),
        scratch_types=[
            pltpu.VMEM((_BPW,), jnp.int32),
            pltpu.VMEM((_CH, 2 * AFL), jnp.float32),
            pltpu.VMEM((_CH, 2 * AFL), jnp.float32),
            pltpu.SemaphoreType.DMA,
            pltpu.SemaphoreType.DMA,
        ],
    )
    def _sc_gather(table_hbm, idx_hbm, out_hbm, idx_v, rows_a, rows_b, sem_a, sem_b):
        wid = lax.axis_index("s") * _NC + lax.axis_index("c")
        base = wid * _BPW
        pltpu.sync_copy(idx_hbm.at[pl.ds(base, _BPW)], idx_v)

        def mk(t, rows, sem):
            return pltpu.make_async_copy(
                table_hbm.at[idx_v.at[pl.ds(t * _CH, _CH)]], rows, sem)

        mk(0, rows_a, sem_a).start()

        def chunk(t, carry):
            even = t % 2 == 0

            @pl.when(even)
            def _():
                mk(t, rows_a, sem_a).wait()

                @pl.when(t + 1 < _NCH)
                def _():
                    mk(t + 1, rows_b, sem_b).start()

                pltpu.sync_copy(rows_a, out_hbm.at[pl.ds(base + t * _CH, _CH)])

            @pl.when(jnp.logical_not(even))
            def _():
                mk(t, rows_b, sem_b).wait()

                @pl.when(t + 1 < _NCH)
                def _():
                    mk(t + 1, rows_a, sem_a).start()

                pltpu.sync_copy(rows_b, out_hbm.at[pl.ds(base + t * _CH, _CH)])

            return carry

        lax.fori_loop(0, _NCH, chunk, 0)

    return _sc_gather


def _stats_body(yng_ref, nbr_ref, x_ref, wedge_ref, wself_ref, b_ref,
                st_ref, gs8, gq8, cs8, ys8, yq8):
    i = pl.program_id(0)

    @pl.when(i == 0)
    def _():
        for r in (gs8, gq8, cs8, ys8, yq8):
            r[...] = jnp.zeros_like(r)

    g = yng_ref[...] + _bdot(nbr_ref[...], wedge_ref[...])
    ys = _bdot(x_ref[...], wself_ref[...]) + b_ref[...]
    gs8[...] += jnp.sum(g.reshape(_BA * 2, 8, 2 * AFL), axis=0)
    gq8[...] += jnp.sum((g * g).reshape(_BA * 2, 8, 2 * AFL), axis=0)
    sg = jnp.sum(g.reshape(_BA, M, 2 * AFL), axis=1)
    cs8[...] += jnp.sum((ys * sg).reshape(_BA // 8, 8, 2 * AFL), axis=0)
    ys8[...] += jnp.sum(ys.reshape(_BA // 8, 8, 2 * AFL), axis=0)
    yq8[...] += jnp.sum((ys * ys).reshape(_BA // 8, 8, 2 * AFL), axis=0)

    @pl.when(i == _NBLK - 1)
    def _():
        ssum = jnp.sum(gs8[...], axis=0, keepdims=True) + M * jnp.sum(
            ys8[...], axis=0, keepdims=True)
        ssq = (jnp.sum(gq8[...], axis=0, keepdims=True)
               + 2.0 * jnp.sum(cs8[...], axis=0, keepdims=True)
               + M * jnp.sum(yq8[...], axis=0, keepdims=True))
        st_ref[0:1, :] = ssum
        st_ref[1:2, :] = ssq


def _conv_stats(yng, nbr2, x, wedge, wself, b):
    return pl.pallas_call(
        _stats_body,
        grid=(_NBLK,),
        in_specs=[
            pl.BlockSpec((_BA * M, 2 * AFL), lambda i: (i, 0)),
            pl.BlockSpec((_BA * M, M), lambda i: (i, 0)),
            pl.BlockSpec((_BA, AFL), lambda i: (i, 0)),
            pl.BlockSpec((M, 2 * AFL), lambda i: (0, 0)),
            pl.BlockSpec((AFL, 2 * AFL), lambda i: (0, 0)),
            pl.BlockSpec((1, 2 * AFL), lambda i: (0, 0)),
        ],
        out_specs=pl.BlockSpec((2, 2 * AFL), lambda i: (0, 0)),
        out_shape=jax.ShapeDtypeStruct((2, 2 * AFL), jnp.float32),
        scratch_shapes=[
            pltpu.VMEM((8, 2 * AFL), jnp.float32),
            pltpu.VMEM((8, 2 * AFL), jnp.float32),
            pltpu.VMEM((8, 2 * AFL), jnp.float32),
            pltpu.VMEM((8, 2 * AFL), jnp.float32),
            pltpu.VMEM((8, 2 * AFL), jnp.float32),
        ],
        compiler_params=pltpu.CompilerParams(
            dimension_semantics=("arbitrary",),
        ),
    )(yng, nbr2, x, wedge, wself, b.reshape(1, 2 * AFL))


def _gate_body(yng_ref, nbr_ref, x_ref, wedge_ref, wself_ref, b_ref,
               st_ref, g1_ref, b1_ref, ns_ref, st2_ref, s28, sq8):
    i = pl.program_id(0)

    @pl.when(i == 0)
    def _():
        s28[...] = jnp.zeros_like(s28)
        sq8[...] = jnp.zeros_like(sq8)

    cnt = float(N * M)
    mu = st_ref[0:1, :] / cnt
    var = st_ref[1:2, :] / cnt - mu * mu
    sc = g1_ref[...] * _rsqrt(var + EPS)
    sh = b1_ref[...] - mu * sc

    ys = _bdot(x_ref[...], wself_ref[...]) + b_ref[...]
    e = _bdot(nbr_ref[...], wedge_ref[...])
    gtd = ((yng_ref[...] + e).reshape(_BA, M, 2 * AFL)
           + ys[:, None, :]).reshape(_BA * M, 2 * AFL)
    t = gtd * sc + sh
    u = jnp.exp(-jnp.abs(t))
    sig = jnp.where(t >= 0.0, 1.0, u) * _recip(1.0 + u)
    tr = pltpu.roll(t, 64, 1)
    sp = jnp.maximum(tr, 0.0) + jnp.log(1.0 + pltpu.roll(u, 64, 1))
    gate = sig * sp
    ns128 = jnp.sum(gate.reshape(_BA, M, 2 * AFL), axis=1)
    ns_ref[...] = ns128[:, :AFL]
    s28[...] += jnp.sum(ns128.reshape(_BA // 8, 8, 2 * AFL), axis=0)
    sq8[...] += jnp.sum((ns128 * ns128).reshape(_BA // 8, 8, 2 * AFL), axis=0)

    @pl.when(i == _NBLK - 1)
    def _():
        st2_ref[0:1, :] = jnp.sum(s28[...], axis=0, keepdims=True)[:, :AFL]
        st2_ref[1:2, :] = jnp.sum(sq8[...], axis=0, keepdims=True)[:, :AFL]


def _conv_gate(yng, nbr2, x, st, wedge, wself, b, g1, b1):
    return pl.pallas_call(
        _gate_body,
        grid=(_NBLK,),
        in_specs=[
            pl.BlockSpec((_BA * M, 2 * AFL), lambda i: (i, 0)),
            pl.BlockSpec((_BA * M, M), lambda i: (i, 0)),
            pl.BlockSpec((_BA, AFL), lambda i: (i, 0)),
            pl.BlockSpec((M, 2 * AFL), lambda i: (0, 0)),
            pl.BlockSpec((AFL, 2 * AFL), lambda i: (0, 0)),
            pl.BlockSpec((1, 2 * AFL), lambda i: (0, 0)),
            pl.BlockSpec((2, 2 * AFL), lambda i: (0, 0)),
            pl.BlockSpec((1, 2 * AFL), lambda i: (0, 0)),
            pl.BlockSpec((1, 2 * AFL), lambda i: (0, 0)),
        ],
        out_specs=[
            pl.BlockSpec((_BA, AFL), lambda i: (i, 0)),
            pl.BlockSpec((2, AFL), lambda i: (0, 0)),
        ],
        out_shape=[
            jax.ShapeDtypeStruct((N, AFL), jnp.float32),
            jax.ShapeDtypeStruct((2, AFL), jnp.float32),
        ],
        scratch_shapes=[
            pltpu.VMEM((8, 2 * AFL), jnp.float32),
            pltpu.VMEM((8, 2 * AFL), jnp.float32),
        ],
        compiler_params=pltpu.CompilerParams(
            dimension_semantics=("arbitrary",),
        ),
    )(yng, nbr2, x, wedge, wself, b.reshape(1, 2 * AFL),
      st, g1.reshape(1, 2 * AFL), b1.reshape(1, 2 * AFL))


def _update_body(x_ref, ns_ref, st2_ref, g2_ref, b2_ref, wn_ref, x_out, yn_out):
    mu = st2_ref[0:1, :] / float(N)
    var = st2_ref[1:2, :] / float(N) - mu * mu
    sc = g2_ref[...] * _rsqrt(var + EPS)
    sh = b2_ref[...] - mu * sc
    x = _softplus(x_ref[...] + ns_ref[...] * sc + sh)
    x_out[...] = x
    yn_out[...] = _bdot(x, wn_ref[...])


def _update_last_body(x_ref, ns_ref, st2_ref, g2_ref, b2_ref, x_out):
    mu = st2_ref[0:1, :] / float(N)
    var = st2_ref[1:2, :] / float(N) - mu * mu
    sc = g2_ref[...] * _rsqrt(var + EPS)
    sh = b2_ref[...] - mu * sc
    x_out[...] = _softplus(x_ref[...] + ns_ref[...] * sc + sh)


def _update(x, ns, st2, g2, b2, wnbr_next):
    in_specs = [
        pl.BlockSpec((_BAC, AFL), lambda i: (i, 0)),
        pl.BlockSpec((_BAC, AFL), lambda i: (i, 0)),
        pl.BlockSpec((2, AFL), lambda i: (0, 0)),
        pl.BlockSpec((1, AFL), lambda i: (0, 0)),
        pl.BlockSpec((1, AFL), lambda i: (0, 0)),
    ]
    args = [x, ns, st2, g2.reshape(1, AFL), b2.reshape(1, AFL)]
    if wnbr_next is None:
        return pl.pallas_call(
            _update_last_body,
            grid=(N // _BAC,),
            in_specs=in_specs,
            out_specs=pl.BlockSpec((_BAC, AFL), lambda i: (i, 0)),
            out_shape=jax.ShapeDtypeStruct((N, AFL), jnp.float32),
        )(*args), None
    return pl.pallas_call(
        _update_body,
        grid=(N // _BAC,),
        in_specs=in_specs + [pl.BlockSpec((AFL, 2 * AFL), lambda i: (0, 0))],
        out_specs=[
            pl.BlockSpec((_BAC, AFL), lambda i: (i, 0)),
            pl.BlockSpec((_BAC, 2 * AFL), lambda i: (i, 0)),
        ],
        out_shape=[
            jax.ShapeDtypeStruct((N, AFL), jnp.float32),
            jax.ShapeDtypeStruct((N, 2 * AFL), jnp.float32),
        ],
    )(*args, wnbr_next)


def _head_body(xr_ref, gl_ref, w1a_ref, w1b_ref, b1_ref, wo_ref, bo_ref, o_ref):
    n0, kk = xr_ref.shape
    per = kk // AFL
    krow = lax.broadcasted_iota(jnp.int32, (kk, AFL), 0) % AFL
    fcol = lax.broadcasted_iota(jnp.int32, (kk, AFL), 1)
    sel = jnp.where(krow == fcol, 1.0 / per, 0.0)
    pooled = jnp.dot(xr_ref[...], sel, preferred_element_type=jnp.float32, precision=lax.Precision.HIGHEST)
    pf = _softplus(pooled)
    gf = _softplus(gl_ref[...])
    h = _bdot(jnp.concatenate([pf, gf], axis=1),
              jnp.concatenate([w1a_ref[...], w1b_ref[...]], axis=0)) + b1_ref[...]
    h = _softplus(h)
    hb = h.astype(jnp.bfloat16).astype(jnp.float32)
    wb = wo_ref[...].astype(jnp.bfloat16).astype(jnp.float32)
    o_ref[...] = jnp.sum(hb * wb, axis=1, keepdims=True) + bo_ref[...]


def _head(x, global_fea, n0, per, w1, b1, wo, bo):
    hfea = w1.shape[1]
    xr = x.reshape(n0, per * AFL)
    return pl.pallas_call(
        _head_body,
        out_shape=jax.ShapeDtypeStruct((n0, 1), jnp.float32),
    )(xr, global_fea, w1[:AFL], w1[AFL:], b1.reshape(1, hfea),
      wo.reshape(1, hfea), bo.reshape(1, 1))


def kernel(atom_fea, nbr_fea, nbr_fea_idx, crystal_atom_idx, atom_type,
           nbr_type, nbr_dist, pair_type, global_fea, params):
    n0, per = crystal_atom_idx.shape
    idx = nbr_fea_idx.reshape(-1).astype(jnp.int32)
    nbr2 = nbr_fea.reshape(N * M, M)
    convs = params["convs"]

    x, yn = _embed(atom_fea, params["emb_W"], params["emb_b"],
                   convs[0]["W"][AFL:2 * AFL])
    for li, p in enumerate(convs):
        w = p["W"]
        yng = _sc_gather_kernel()(yn, idx)
        st = _conv_stats(yng, nbr2, x, w[2 * AFL:], w[:AFL], p["b"])
        ns, st2 = _conv_gate(yng, nbr2, x, st, w[2 * AFL:], w[:AFL], p["b"],
                             p["bn1_g"], p["bn1_b"])
        wnbr_next = (convs[li + 1]["W"][AFL:2 * AFL]
                     if li + 1 < len(convs) else None)
        x, yn = _update(x, ns, st2, p["bn2_g"], p["bn2_b"], wnbr_next)

    return _head(x, global_fea, n0, per,
                 params["fc1_W"], params["fc1_b"],
                 params["out_W"], params["out_b"])


# speedup vs baseline: 2.1198x; 1.0604x over previous
import functools

import jax
import jax.numpy as jnp
from jax import lax
from jax.experimental import pallas as pl
from jax.experimental.pallas import tpu as pltpu
from jax.experimental.pallas import tpu_sc as plsc

N = 50000
M = 16
AFL = 64
EPS = 1e-5

_NC = 2
_NS = 16
_NW = _NC * _NS

_BA = 1000
_NBLK = N // _BA
_BAC = 5000


def _softplus(x):
    return jnp.maximum(x, 0.0) + jnp.log1p(jnp.exp(-jnp.abs(x)))


def _rsqrt(x):
    r = lax.rsqrt(x)
    return r * (1.5 - 0.5 * x * r * r)


def _recip(d):
    r = 1.0 / d
    return r * (2.0 - d * r)


def _bdot(a, b):
    return jnp.dot(a.astype(jnp.bfloat16), b.astype(jnp.bfloat16),
                   preferred_element_type=jnp.float32)


def _emb_body(a_ref, w_ref, b_ref, wn_ref, x_ref, yn_ref):
    x = _bdot(a_ref[...], w_ref[...]) + b_ref[...]
    x_ref[...] = x
    yn_ref[...] = _bdot(x, wn_ref[...])


def _embed(atom_fea, w, b, wnbr):
    return pl.pallas_call(
        _emb_body,
        grid=(N // _BAC,),
        in_specs=[
            pl.BlockSpec((_BAC, 128), lambda i: (i, 0)),
            pl.BlockSpec((128, AFL), lambda i: (0, 0)),
            pl.BlockSpec((1, AFL), lambda i: (0, 0)),
            pl.BlockSpec((AFL, 2 * AFL), lambda i: (0, 0)),
        ],
        out_specs=[
            pl.BlockSpec((_BAC, AFL), lambda i: (i, 0)),
            pl.BlockSpec((_BAC, 2 * AFL), lambda i: (i, 0)),
        ],
        out_shape=[
            jax.ShapeDtypeStruct((N, AFL), jnp.float32),
            jax.ShapeDtypeStruct((N, 2 * AFL), jnp.float32),
        ],
    )(atom_fea, w, b.reshape(1, AFL), wnbr)


_B_TOT = N * M
_BPW = _B_TOT // _NW
_CH = 200
_NCH = _BPW // _CH


@functools.cache
def _sc_gather_kernel():
    @functools.partial(
        pl.kernel,
        mesh=plsc.VectorSubcoreMesh(core_axis_name="c", subcore_axis_name="s"),
        out_type=jax.ShapeDtypeStruct((_B_TOT, 2 * AFL), jnp.float32),
        scratch_types=[
            pltpu.VMEM((_BPW,), jnp.int32),
            pltpu.VMEM((_CH, 2 * AFL), jnp.float32),
            pltpu.VMEM((_CH, 2 * AFL), jnp.float32),
            pltpu.SemaphoreType.DMA,
            pltpu.SemaphoreType.DMA,
        ],
    )
    def _sc_gather(table_hbm, idx_hbm, out_hbm, idx_v, rows_a, rows_b, sem_a, sem_b):
        wid = lax.axis_index("s") * _NC + lax.axis_index("c")
        base = wid * _BPW
        pltpu.sync_copy(idx_hbm.at[pl.ds(base, _BPW)], idx_v)

        def mk(t, rows, sem):
            return pltpu.make_async_copy(
                table_hbm.at[idx_v.at[pl.ds(t * _CH, _CH)]], rows, sem)

        mk(0, rows_a, sem_a).start()

        def chunk(t, carry):
            even = t % 2 == 0

            @pl.when(even)
            def _():
                mk(t, rows_a, sem_a).wait()

                @pl.when(t + 1 < _NCH)
                def _():
                    mk(t + 1, rows_b, sem_b).start()

                pltpu.sync_copy(rows_a, out_hbm.at[pl.ds(base + t * _CH, _CH)])

            @pl.when(jnp.logical_not(even))
            def _():
                mk(t, rows_b, sem_b).wait()

                @pl.when(t + 1 < _NCH)
                def _():
                    mk(t + 1, rows_a, sem_a).start()

                pltpu.sync_copy(rows_b, out_hbm.at[pl.ds(base + t * _CH, _CH)])

            return carry

        lax.fori_loop(0, _NCH, chunk, 0)

    return _sc_gather


def _stats_body(yng_ref, nbr_ref, x_ref, wedge_ref, wself_ref, b_ref,
                st_ref, gs8, gq8, cs8, ys8, yq8):
    i = pl.program_id(0)

    @pl.when(i == 0)
    def _():
        for r in (gs8, gq8, cs8, ys8, yq8):
            r[...] = jnp.zeros_like(r)

    g = yng_ref[...] + _bdot(nbr_ref[...], wedge_ref[...])
    ys = _bdot(x_ref[...], wself_ref[...]) + b_ref[...]
    gs8[...] += jnp.sum(g.reshape(_BA * 2, 8, 2 * AFL), axis=0)
    gq8[...] += jnp.sum((g * g).reshape(_BA * 2, 8, 2 * AFL), axis=0)
    sg = jnp.sum(g.reshape(_BA, M, 2 * AFL), axis=1)
    cs8[...] += jnp.sum((ys * sg).reshape(_BA // 8, 8, 2 * AFL), axis=0)
    ys8[...] += jnp.sum(ys.reshape(_BA // 8, 8, 2 * AFL), axis=0)
    yq8[...] += jnp.sum((ys * ys).reshape(_BA // 8, 8, 2 * AFL), axis=0)

    @pl.when(i == _NBLK - 1)
    def _():
        ssum = jnp.sum(gs8[...], axis=0, keepdims=True) + M * jnp.sum(
            ys8[...], axis=0, keepdims=True)
        ssq = (jnp.sum(gq8[...], axis=0, keepdims=True)
               + 2.0 * jnp.sum(cs8[...], axis=0, keepdims=True)
               + M * jnp.sum(yq8[...], axis=0, keepdims=True))
        st_ref[0:1, :] = ssum
        st_ref[1:2, :] = ssq


def _conv_stats(yng, nbr2, x, wedge, wself, b):
    return pl.pallas_call(
        _stats_body,
        grid=(_NBLK,),
        in_specs=[
            pl.BlockSpec((_BA * M, 2 * AFL), lambda i: (i, 0)),
            pl.BlockSpec((_BA * M, M), lambda i: (i, 0)),
            pl.BlockSpec((_BA, AFL), lambda i: (i, 0)),
            pl.BlockSpec((M, 2 * AFL), lambda i: (0, 0)),
            pl.BlockSpec((AFL, 2 * AFL), lambda i: (0, 0)),
            pl.BlockSpec((1, 2 * AFL), lambda i: (0, 0)),
        ],
        out_specs=pl.BlockSpec((2, 2 * AFL), lambda i: (0, 0)),
        out_shape=jax.ShapeDtypeStruct((2, 2 * AFL), jnp.float32),
        scratch_shapes=[
            pltpu.VMEM((8, 2 * AFL), jnp.float32),
            pltpu.VMEM((8, 2 * AFL), jnp.float32),
            pltpu.VMEM((8, 2 * AFL), jnp.float32),
            pltpu.VMEM((8, 2 * AFL), jnp.float32),
            pltpu.VMEM((8, 2 * AFL), jnp.float32),
        ],
        compiler_params=pltpu.CompilerParams(
            dimension_semantics=("arbitrary",),
        ),
    )(yng, nbr2, x, wedge, wself, b.reshape(1, 2 * AFL))


def _gate_body(yng_ref, nbr_ref, x_ref, wedge_ref, wself_ref, b_ref,
               st_ref, g1_ref, b1_ref, ns_ref, st2_ref, s28, sq8):
    i = pl.program_id(0)

    @pl.when(i == 0)
    def _():
        s28[...] = jnp.zeros_like(s28)
        sq8[...] = jnp.zeros_like(sq8)

    cnt = float(N * M)
    mu = st_ref[0:1, :] / cnt
    var = st_ref[1:2, :] / cnt - mu * mu
    sc = g1_ref[...] * _rsqrt(var + EPS)
    sh = b1_ref[...] - mu * sc

    ys = _bdot(x_ref[...], wself_ref[...]) + b_ref[...]
    e = _bdot(nbr_ref[...], wedge_ref[...])
    gtd = ((yng_ref[...] + e).reshape(_BA, M, 2 * AFL)
           + ys[:, None, :]).reshape(_BA * M, 2 * AFL)
    t = gtd * sc + sh
    u = jnp.exp(-jnp.abs(t))
    sig = jnp.where(t >= 0.0, 1.0, u) * _recip(1.0 + u)
    sp = jnp.maximum(t, 0.0) + jnp.log(1.0 + u)
    gate = sig * pltpu.roll(sp, 64, 1)
    ns128 = jnp.sum(gate.reshape(_BA, M, 2 * AFL), axis=1)
    ns_ref[...] = ns128[:, :AFL]
    s28[...] += jnp.sum(ns128.reshape(_BA // 8, 8, 2 * AFL), axis=0)
    sq8[...] += jnp.sum((ns128 * ns128).reshape(_BA // 8, 8, 2 * AFL), axis=0)

    @pl.when(i == _NBLK - 1)
    def _():
        st2_ref[0:1, :] = jnp.sum(s28[...], axis=0, keepdims=True)[:, :AFL]
        st2_ref[1:2, :] = jnp.sum(sq8[...], axis=0, keepdims=True)[:, :AFL]


def _conv_gate(yng, nbr2, x, st, wedge, wself, b, g1, b1):
    return pl.pallas_call(
        _gate_body,
        grid=(_NBLK,),
        in_specs=[
            pl.BlockSpec((_BA * M, 2 * AFL), lambda i: (i, 0)),
            pl.BlockSpec((_BA * M, M), lambda i: (i, 0)),
            pl.BlockSpec((_BA, AFL), lambda i: (i, 0)),
            pl.BlockSpec((M, 2 * AFL), lambda i: (0, 0)),
            pl.BlockSpec((AFL, 2 * AFL), lambda i: (0, 0)),
            pl.BlockSpec((1, 2 * AFL), lambda i: (0, 0)),
            pl.BlockSpec((2, 2 * AFL), lambda i: (0, 0)),
            pl.BlockSpec((1, 2 * AFL), lambda i: (0, 0)),
            pl.BlockSpec((1, 2 * AFL), lambda i: (0, 0)),
        ],
        out_specs=[
            pl.BlockSpec((_BA, AFL), lambda i: (i, 0)),
            pl.BlockSpec((2, AFL), lambda i: (0, 0)),
        ],
        out_shape=[
            jax.ShapeDtypeStruct((N, AFL), jnp.float32),
            jax.ShapeDtypeStruct((2, AFL), jnp.float32),
        ],
        scratch_shapes=[
            pltpu.VMEM((8, 2 * AFL), jnp.float32),
            pltpu.VMEM((8, 2 * AFL), jnp.float32),
        ],
        compiler_params=pltpu.CompilerParams(
            dimension_semantics=("arbitrary",),
        ),
    )(yng, nbr2, x, wedge, wself, b.reshape(1, 2 * AFL),
      st, g1.reshape(1, 2 * AFL), b1.reshape(1, 2 * AFL))


def _update_body(x_ref, ns_ref, st2_ref, g2_ref, b2_ref, wn_ref, x_out, yn_out):
    mu = st2_ref[0:1, :] / float(N)
    var = st2_ref[1:2, :] / float(N) - mu * mu
    sc = g2_ref[...] * _rsqrt(var + EPS)
    sh = b2_ref[...] - mu * sc
    x = _softplus(x_ref[...] + ns_ref[...] * sc + sh)
    x_out[...] = x
    yn_out[...] = _bdot(x, wn_ref[...])


def _update_last_body(x_ref, ns_ref, st2_ref, g2_ref, b2_ref, x_out):
    mu = st2_ref[0:1, :] / float(N)
    var = st2_ref[1:2, :] / float(N) - mu * mu
    sc = g2_ref[...] * _rsqrt(var + EPS)
    sh = b2_ref[...] - mu * sc
    x_out[...] = _softplus(x_ref[...] + ns_ref[...] * sc + sh)


def _update(x, ns, st2, g2, b2, wnbr_next):
    in_specs = [
        pl.BlockSpec((_BAC, AFL), lambda i: (i, 0)),
        pl.BlockSpec((_BAC, AFL), lambda i: (i, 0)),
        pl.BlockSpec((2, AFL), lambda i: (0, 0)),
        pl.BlockSpec((1, AFL), lambda i: (0, 0)),
        pl.BlockSpec((1, AFL), lambda i: (0, 0)),
    ]
    args = [x, ns, st2, g2.reshape(1, AFL), b2.reshape(1, AFL)]
    if wnbr_next is None:
        return pl.pallas_call(
            _update_last_body,
            grid=(N // _BAC,),
            in_specs=in_specs,
            out_specs=pl.BlockSpec((_BAC, AFL), lambda i: (i, 0)),
            out_shape=jax.ShapeDtypeStruct((N, AFL), jnp.float32),
        )(*args), None
    return pl.pallas_call(
        _update_body,
        grid=(N // _BAC,),
        in_specs=in_specs + [pl.BlockSpec((AFL, 2 * AFL), lambda i: (0, 0))],
        out_specs=[
            pl.BlockSpec((_BAC, AFL), lambda i: (i, 0)),
            pl.BlockSpec((_BAC, 2 * AFL), lambda i: (i, 0)),
        ],
        out_shape=[
            jax.ShapeDtypeStruct((N, AFL), jnp.float32),
            jax.ShapeDtypeStruct((N, 2 * AFL), jnp.float32),
        ],
    )(*args, wnbr_next)


def _head_body(xr_ref, gl_ref, w1a_ref, w1b_ref, b1_ref, wo_ref, bo_ref, o_ref):
    n0, kk = xr_ref.shape
    per = kk // AFL
    krow = lax.broadcasted_iota(jnp.int32, (kk, AFL), 0) % AFL
    fcol = lax.broadcasted_iota(jnp.int32, (kk, AFL), 1)
    sel = jnp.where(krow == fcol, 1.0 / per, 0.0)
    pooled = jnp.dot(xr_ref[...], sel, preferred_element_type=jnp.float32, precision=lax.Precision.HIGHEST)
    pf = _softplus(pooled)
    gf = _softplus(gl_ref[...])
    h = _bdot(jnp.concatenate([pf, gf], axis=1),
              jnp.concatenate([w1a_ref[...], w1b_ref[...]], axis=0)) + b1_ref[...]
    h = _softplus(h)
    hb = h.astype(jnp.bfloat16).astype(jnp.float32)
    wb = wo_ref[...].astype(jnp.bfloat16).astype(jnp.float32)
    o_ref[...] = jnp.sum(hb * wb, axis=1, keepdims=True) + bo_ref[...]


def _head(x, global_fea, n0, per, w1, b1, wo, bo):
    hfea = w1.shape[1]
    xr = x.reshape(n0, per * AFL)
    return pl.pallas_call(
        _head_body,
        out_shape=jax.ShapeDtypeStruct((n0, 1), jnp.float32),
    )(xr, global_fea, w1[:AFL], w1[AFL:], b1.reshape(1, hfea),
      wo.reshape(1, hfea), bo.reshape(1, 1))


def kernel(atom_fea, nbr_fea, nbr_fea_idx, crystal_atom_idx, atom_type,
           nbr_type, nbr_dist, pair_type, global_fea, params):
    n0, per = crystal_atom_idx.shape
    idx = nbr_fea_idx.reshape(-1).astype(jnp.int32)
    nbr2 = nbr_fea.reshape(N * M, M)
    convs = params["convs"]

    x, yn = _embed(atom_fea, params["emb_W"], params["emb_b"],
                   convs[0]["W"][AFL:2 * AFL])
    for li, p in enumerate(convs):
        w = p["W"]
        yng = _sc_gather_kernel()(yn, idx)
        st = _conv_stats(yng, nbr2, x, w[2 * AFL:], w[:AFL], p["b"])
        ns, st2 = _conv_gate(yng, nbr2, x, st, w[2 * AFL:], w[:AFL], p["b"],
                             p["bn1_g"], p["bn1_b"])
        wnbr_next = (convs[li + 1]["W"][AFL:2 * AFL]
                     if li + 1 < len(convs) else None)
        x, yn = _update(x, ns, st2, p["bn2_g"], p["bn2_b"], wnbr_next)

    return _head(x, global_fea, n0, per,
                 params["fc1_W"], params["fc1_b"],
                 params["out_W"], params["out_b"])
